# baseline (speedup 1.0000x reference)
import functools

import jax
import jax.numpy as jnp
from jax import lax
from jax.experimental import pallas as pl
from jax.experimental.pallas import tpu as pltpu
from jax.experimental.pallas import tpu_sc as plsc

B, S, F = 2, 8192, 2048
L = 16
NC, NS = 2, 16
NW = NC * NS
GPW = 8
FW = GPW * L
WPB = NW // B
CHUNK = 256
NCHUNK = S // CHUNK
NBUF = 3

_mesh = plsc.VectorSubcoreMesh(core_axis_name="c", subcore_axis_name="s")


@functools.partial(
    pl.kernel,
    mesh=_mesh,
    out_type=jax.ShapeDtypeStruct((B, S, F), jnp.float32),
    compiler_params=pltpu.CompilerParams(
        use_tc_tiling_on_sc=False, needs_layout_passes=False),
    scratch_types=(
        [pltpu.VMEM((NBUF, CHUNK, FW), jnp.float32)]
        + [pltpu.SemaphoreType.DMA] * (2 * NBUF)
    ),
)
def _cumsum_sc(x_hbm, out_hbm, buf, *sems):
    in_sems, out_sems = sems[:NBUF], sems[NBUF:]
    wid = lax.axis_index("s") * NC + lax.axis_index("c")
    b = wid // WPB
    f0 = (wid % WPB) * FW

    def src(ci):
        return x_hbm.at[b, pl.ds(ci * CHUNK, CHUNK), pl.ds(f0, FW)]

    def dst(ci):
        return out_hbm.at[b, pl.ds(ci * CHUNK, CHUNK), pl.ds(f0, FW)]

    for ci in range(NBUF - 1):
        pltpu.async_copy(src(ci), buf.at[ci % NBUF], in_sems[ci % NBUF])

    accs = tuple(jnp.zeros((L,), jnp.float32) for _ in range(GPW))
    for ci in range(NCHUNK):
        k = ci % NBUF
        pltpu.make_async_copy(src(ci), buf.at[k], in_sems[k]).wait()

        @plsc.parallel_loop(0, CHUNK, 1, unroll=8, carry=accs)
        def row_body(i, accs, k=k):
            new = []
            for g in range(GPW):
                v = buf[k, i, pl.ds(g * L, L)]
                buf[k, i, pl.ds(g * L, L)] = accs[g]
                new.append(accs[g] + v)
            return tuple(new)

        accs = row_body
        pltpu.async_copy(buf.at[k], dst(ci), out_sems[k])
        nci = ci + (NBUF - 1)
        if nci < NCHUNK:
            nk = nci % NBUF
            if nci - NBUF >= 0:
                pltpu.make_async_copy(
                    buf.at[nk], dst(nci - NBUF), out_sems[nk]).wait()
            pltpu.async_copy(src(nci), buf.at[nk], in_sems[nk])

    for ci in range(NCHUNK - NBUF, NCHUNK):
        k = ci % NBUF
        pltpu.make_async_copy(buf.at[k], dst(ci), out_sems[k]).wait()


def kernel(x):
    return _cumsum_sc(x)


# speedup vs baseline: 2.8983x; 2.8983x over previous
import functools

import jax
import jax.numpy as jnp
from jax import lax
from jax.experimental import pallas as pl
from jax.experimental.pallas import tpu as pltpu
from jax.experimental.pallas import tpu_sc as plsc

B, S, F = 2, 8192, 2048
L = 16
NC, NS = 2, 16
NW = NC * NS
GPW = 8
FW = GPW * L
WPB = NW // B
CHUNK = 256
NCHUNK = S // CHUNK
NBUF = 3

_mesh = plsc.VectorSubcoreMesh(core_axis_name="c", subcore_axis_name="s")


@functools.partial(
    pl.kernel,
    mesh=_mesh,
    out_type=jax.ShapeDtypeStruct((B, S, F), jnp.float32),
    compiler_params=pltpu.CompilerParams(
        use_tc_tiling_on_sc=True, needs_layout_passes=False),
    scratch_types=(
        [pltpu.VMEM((NBUF, CHUNK, FW), jnp.float32)]
        + [pltpu.SemaphoreType.DMA] * (2 * NBUF)
    ),
)
def _cumsum_sc(x_hbm, out_hbm, buf, *sems):
    in_sems, out_sems = sems[:NBUF], sems[NBUF:]
    wid = lax.axis_index("s") * NC + lax.axis_index("c")
    b = wid // WPB
    f0 = (wid % WPB) * FW

    def src(ci):
        return x_hbm.at[b, pl.ds(ci * CHUNK, CHUNK), pl.ds(f0, FW)]

    def dst(ci):
        return out_hbm.at[b, pl.ds(ci * CHUNK, CHUNK), pl.ds(f0, FW)]

    for ci in range(NBUF - 1):
        pltpu.async_copy(src(ci), buf.at[ci % NBUF], in_sems[ci % NBUF])

    accs = tuple(jnp.zeros((L,), jnp.float32) for _ in range(GPW))
    for ci in range(NCHUNK):
        k = ci % NBUF
        pltpu.make_async_copy(src(ci), buf.at[k], in_sems[k]).wait()

        @plsc.parallel_loop(0, CHUNK, 1, unroll=8, carry=accs)
        def row_body(i, accs, k=k):
            new = []
            for g in range(GPW):
                v = buf[k, i, pl.ds(g * L, L)]
                buf[k, i, pl.ds(g * L, L)] = accs[g]
                new.append(accs[g] + v)
            return tuple(new)

        accs = row_body
        pltpu.async_copy(buf.at[k], dst(ci), out_sems[k])
        nci = ci + (NBUF - 1)
        if nci < NCHUNK:
            nk = nci % NBUF
            if nci - NBUF >= 0:
                pltpu.make_async_copy(
                    buf.at[nk], dst(nci - NBUF), out_sems[nk]).wait()
            pltpu.async_copy(src(nci), buf.at[nk], in_sems[nk])

    for ci in range(NCHUNK - NBUF, NCHUNK):
        k = ci % NBUF
        pltpu.make_async_copy(buf.at[k], dst(ci), out_sems[k]).wait()


def kernel(x):
    return _cumsum_sc(x)


# speedup vs baseline: 2.9196x; 1.0073x over previous
import functools

import jax
import jax.numpy as jnp
from jax import lax
from jax.experimental import pallas as pl
from jax.experimental.pallas import tpu as pltpu
from jax.experimental.pallas import tpu_sc as plsc

B, S, F = 2, 8192, 2048
L = 16
NC, NS = 2, 16
NW = NC * NS
GPW = 8
FW = GPW * L
WPB = NW // B
CHUNK = 256
NCHUNK = S // CHUNK
NBUF = 3

_mesh = plsc.VectorSubcoreMesh(core_axis_name="c", subcore_axis_name="s")


@functools.partial(
    pl.kernel,
    mesh=_mesh,
    out_type=jax.ShapeDtypeStruct((B, S, F), jnp.float32),
    compiler_params=pltpu.CompilerParams(
        use_tc_tiling_on_sc=True, needs_layout_passes=False),
    scratch_types=(
        [pltpu.VMEM((NBUF, CHUNK, FW), jnp.float32)]
        + [pltpu.SemaphoreType.DMA] * (2 * NBUF)
    ),
)
def _cumsum_sc(x_hbm, out_hbm, buf, *sems):
    in_sems, out_sems = sems[:NBUF], sems[NBUF:]
    wid = lax.axis_index("s") * NC + lax.axis_index("c")
    b = wid // WPB
    f0 = (wid % WPB) * FW

    def src(ci):
        return x_hbm.at[b, pl.ds(ci * CHUNK, CHUNK), pl.ds(f0, FW)]

    def dst(ci):
        return out_hbm.at[b, pl.ds(ci * CHUNK, CHUNK), pl.ds(f0, FW)]

    for ci in range(NBUF - 1):
        pltpu.async_copy(src(ci), buf.at[ci % NBUF], in_sems[ci % NBUF])

    accs = tuple(jnp.zeros((L,), jnp.float32) for _ in range(GPW))
    for ci in range(NCHUNK):
        k = ci % NBUF
        pltpu.make_async_copy(src(ci), buf.at[k], in_sems[k]).wait()

        @plsc.parallel_loop(0, CHUNK, 1, unroll=16, carry=accs)
        def row_body(i, accs, k=k):
            new = []
            for g in range(GPW):
                v = buf[k, i, pl.ds(g * L, L)]
                buf[k, i, pl.ds(g * L, L)] = accs[g]
                new.append(accs[g] + v)
            return tuple(new)

        accs = row_body
        pltpu.async_copy(buf.at[k], dst(ci), out_sems[k])
        nci = ci + (NBUF - 1)
        if nci < NCHUNK:
            nk = nci % NBUF
            if nci - NBUF >= 0:
                pltpu.make_async_copy(
                    buf.at[nk], dst(nci - NBUF), out_sems[nk]).wait()
            pltpu.async_copy(src(nci), buf.at[nk], in_sems[nk])

    for ci in range(NCHUNK - NBUF, NCHUNK):
        k = ci % NBUF
        pltpu.make_async_copy(buf.at[k], dst(ci), out_sems[k]).wait()


def kernel(x):
    return _cumsum_sc(x)


# speedup vs baseline: 3.1158x; 1.0672x over previous
import functools

import jax
import jax.numpy as jnp
from jax import lax
from jax.experimental import pallas as pl
from jax.experimental.pallas import tpu as pltpu
from jax.experimental.pallas import tpu_sc as plsc

B, S, F = 2, 8192, 2048
L = 16
NC, NS = 2, 16
NW = NC * NS
GPW = 8
FW = GPW * L
WPB = NW // B
CHUNK = 64
NCHUNK = S // CHUNK
NBUF = 4
NROUND = NCHUNK // NBUF

_mesh = plsc.VectorSubcoreMesh(core_axis_name="c", subcore_axis_name="s")


@functools.partial(
    pl.kernel,
    mesh=_mesh,
    out_type=jax.ShapeDtypeStruct((B, S, F), jnp.float32),
    compiler_params=pltpu.CompilerParams(
        use_tc_tiling_on_sc=True, needs_layout_passes=False),
    scratch_types=(
        [pltpu.VMEM((NBUF, CHUNK, FW), jnp.float32)]
        + [pltpu.SemaphoreType.DMA] * (2 * NBUF)
    ),
)
def _cumsum_sc(x_hbm, out_hbm, buf, *sems):
    in_sems, out_sems = sems[:NBUF], sems[NBUF:]
    wid = lax.axis_index("s") * NC + lax.axis_index("c")
    b = wid // WPB
    f0 = (wid % WPB) * FW

    def src(ci):
        return x_hbm.at[b, pl.ds(ci * CHUNK, CHUNK), pl.ds(f0, FW)]

    def dst(ci):
        return out_hbm.at[b, pl.ds(ci * CHUNK, CHUNK), pl.ds(f0, FW)]

    def chunk_body(ci, j, accs, first, last):
        pltpu.make_async_copy(src(ci), buf.at[j], in_sems[j]).wait()

        @plsc.parallel_loop(0, CHUNK, 1, unroll=16, carry=accs)
        def row_body(i, accs, j=j):
            new = []
            for g in range(GPW):
                v = buf[j, i, pl.ds(g * L, L)]
                buf[j, i, pl.ds(g * L, L)] = accs[g]
                new.append(accs[g] + v)
            return tuple(new)

        accs = row_body
        pltpu.async_copy(buf.at[j], dst(ci), out_sems[j])
        if not last:
            nj = (j + NBUF - 1) % NBUF
            if not (first and j == 0):
                pltpu.make_async_copy(
                    buf.at[nj], dst(ci - 1), out_sems[nj]).wait()
            pltpu.async_copy(src(ci + NBUF - 1), buf.at[nj], in_sems[nj])
        return accs

    for ci in range(NBUF - 1):
        pltpu.async_copy(src(ci), buf.at[ci], in_sems[ci])

    accs = tuple(jnp.zeros((L,), jnp.float32) for _ in range(GPW))

    for j in range(NBUF):
        accs = chunk_body(j, j, accs, first=True, last=False)

    def round_body(r, accs):
        for j in range(NBUF):
            accs = chunk_body(r * NBUF + j, j, accs, first=False, last=False)
        return accs

    accs = lax.fori_loop(1, NROUND - 1, round_body, accs)

    for j in range(NBUF):
        ci = NCHUNK - NBUF + j
        accs = chunk_body(ci, j, accs, first=False, last=(j > 0))

    for j in range(NBUF):
        pltpu.make_async_copy(
            buf.at[j], dst(NCHUNK - NBUF + j), out_sems[j]).wait()


def kernel(x):
    return _cumsum_sc(x)
